# Initial kernel scaffold; baseline (speedup 1.0000x reference)
import jax
import jax.numpy as jnp
from jax.experimental import pallas as pl


def kernel(x, y):
    raise NotImplementedError("write your pallas kernel here")


# speedup vs baseline: 1.0233x; 1.0233x over previous
import jax
import jax.numpy as jnp
from jax.experimental import pallas as pl


_ROWS_PER_BLOCK = 1024


def _body(x_ref, y_ref, o_ref):
    o_ref[...] = x_ref[...]

    @pl.when(pl.program_id(0) == 0)
    def _patch():
        col = jax.lax.broadcasted_iota(jnp.int32, (1, 128), 1)
        o_ref[0:1, :] = jnp.where(col == 3, y_ref[0, 2], x_ref[0:1, :])


def kernel(x, y):
    n_rows, n_cols = x.shape
    grid = (n_rows // _ROWS_PER_BLOCK,)
    return pl.pallas_call(
        _body,
        grid=grid,
        in_specs=[
            pl.BlockSpec((_ROWS_PER_BLOCK, n_cols), lambda i: (i, 0)),
            pl.BlockSpec((8, n_cols), lambda i: (0, 0)),
        ],
        out_specs=pl.BlockSpec((_ROWS_PER_BLOCK, n_cols), lambda i: (i, 0)),
        out_shape=jax.ShapeDtypeStruct(x.shape, x.dtype),
    )(x, y)
